# baseline (speedup 1.0000x reference)
import functools

import jax
import jax.numpy as jnp
from jax.experimental import pallas as pl
from jax.experimental.pallas import tpu as pltpu

_S = 64
_NEG = -1e30


def _body(x_ref, b_ref, w1_ref, b1_ref, w2_ref, b2_ref, out_ref,
          m_ref, d_ref, gt_ref, xp_ref, *, nsteps, n_total, bn, s, padded):
    i = pl.program_id(0)

    @pl.when(i == 0)
    def _init():
        m_ref[:] = jnp.full((s, 1), _NEG, jnp.float32)
        d_ref[:] = jnp.zeros((s, 1), jnp.float32)
        out_ref[:] = jnp.zeros_like(out_ref)

    xb = x_ref[:].astype(jnp.bfloat16)
    h = jnp.maximum(
        jnp.dot(xb, w1_ref[:], preferred_element_type=jnp.float32)
        + b1_ref[:], 0.0).astype(jnp.bfloat16)
    gt_new = jax.lax.dot_general(w2_ref[:], h, (((1,), (1,)), ((), ())),
                                 preferred_element_type=jnp.float32)
    gt_new = gt_new + b2_ref[0, 0]

    @pl.when(i == 0)
    def _prime():
        xp_ref[:] = xb

    gt = gt_ref[:]
    xp = xp_ref[:]
    b_ids = b_ref[:].reshape(1, bn)
    seg = jax.lax.broadcasted_iota(jnp.int32, (s, bn), 0)
    pt = jnp.logical_and(seg == b_ids, i > 0)
    if padded:
        col = (i - 1) * bn + jax.lax.broadcasted_iota(jnp.int32, (s, bn), 1)
        pt = jnp.logical_and(pt, col < n_total)

    m_old = m_ref[:]
    blk_max = jnp.max(jnp.where(pt, gt, _NEG), axis=1, keepdims=True)
    m_new = jnp.maximum(m_old, blk_max)
    scale = jnp.exp(m_old - m_new)
    e = jnp.where(pt, jnp.exp(gt - m_new), 0.0)
    d_ref[:] = d_ref[:] * scale + jnp.sum(e, axis=1, keepdims=True)
    m_ref[:] = m_new
    out_ref[:] = (out_ref[:] * scale
                  + jnp.dot(e.astype(jnp.bfloat16), xp,
                            preferred_element_type=jnp.float32))

    gt_ref[:] = gt_new
    xp_ref[:] = xb

    @pl.when(i == nsteps)
    def _fin():
        out_ref[:] = out_ref[:] / (d_ref[:] + 1e-16)


def kernel(inputs, batch, W1, b1, W2, b2):
    n, d_dim = inputs.shape
    h_dim = W1.shape[1]
    s = _S
    bn = 4000 if n % 4000 == 0 else (2000 if n % 2000 == 0 else 1000)
    pad = (-n) % bn
    if pad:
        inputs = jnp.pad(inputs, ((0, pad), (0, 0)))
        batch = jnp.pad(batch, (0, pad))
    nsteps = (n + pad) // bn

    batch3 = batch.astype(jnp.int32).reshape(nsteps, 1, bn)
    b1r = b1.reshape(1, h_dim).astype(jnp.bfloat16)
    w2r = W2.reshape(1, h_dim).astype(jnp.bfloat16)
    b2r = b2.reshape(1, 1)
    W1 = W1.astype(jnp.bfloat16)
    last = nsteps - 1

    out = pl.pallas_call(
        functools.partial(_body, nsteps=nsteps, n_total=n, bn=bn, s=s,
                          padded=bool(pad)),
        grid=(nsteps + 1,),
        in_specs=[
            pl.BlockSpec((bn, d_dim), lambda i: (jnp.minimum(i, last), 0)),
            pl.BlockSpec((1, 1, bn),
                         lambda i: (jnp.maximum(i - 1, 0), 0, 0)),
            pl.BlockSpec((d_dim, h_dim), lambda i: (0, 0)),
            pl.BlockSpec((1, h_dim), lambda i: (0, 0)),
            pl.BlockSpec((1, h_dim), lambda i: (0, 0)),
            pl.BlockSpec((1, 1), lambda i: (0, 0)),
        ],
        out_specs=pl.BlockSpec((s, d_dim), lambda i: (0, 0)),
        out_shape=jax.ShapeDtypeStruct((s, d_dim), jnp.float32),
        scratch_shapes=[pltpu.VMEM((s, 1), jnp.float32),
                        pltpu.VMEM((s, 1), jnp.float32),
                        pltpu.VMEM((1, bn), jnp.float32),
                        pltpu.VMEM((bn, d_dim), jnp.bfloat16)],
        compiler_params=pltpu.CompilerParams(
            dimension_semantics=("arbitrary",)),
    )(inputs, batch3, W1, b1r, w2r, b2r)
    return out


# speedup vs baseline: 1.1029x; 1.1029x over previous
import functools

import jax
import jax.numpy as jnp
from jax.experimental import pallas as pl
from jax.experimental.pallas import tpu as pltpu

_S = 64
_NEG = -1e30


def _body(x_hbm, b_ref, w1_ref, b1_ref, w2_ref, b2_ref, out_ref,
          m_ref, d_ref, xbuf, sems, *, nsteps, n_total, bn, s, padded):
    i = pl.program_id(0)

    def x_copy(blk, slot):
        return pltpu.make_async_copy(
            x_hbm.at[pl.ds(blk * bn, bn), :], xbuf.at[slot], sems.at[slot])

    @pl.when(i == 0)
    def _init():
        m_ref[:] = jnp.full((s, 1), _NEG, jnp.float32)
        d_ref[:] = jnp.zeros((s, 1), jnp.float32)
        out_ref[:] = jnp.zeros_like(out_ref)
        x_copy(0, 0).start()

    @pl.when(i + 1 < nsteps)
    def _prefetch():
        x_copy(i + 1, (i + 1) % 2).start()

    x_copy(i, i % 2).wait()
    xb = xbuf[i % 2].astype(jnp.bfloat16)
    h = jnp.maximum(
        jnp.dot(xb, w1_ref[:], preferred_element_type=jnp.float32)
        + b1_ref[:], 0.0).astype(jnp.bfloat16)
    gt = jax.lax.dot_general(w2_ref[:], h, (((1,), (1,)), ((), ())),
                             preferred_element_type=jnp.float32)
    gt = gt + b2_ref[0, 0]

    b_ids = b_ref[:].reshape(1, bn)
    seg = jax.lax.broadcasted_iota(jnp.int32, (s, bn), 0)
    pt = seg == b_ids
    if padded:
        col = i * bn + jax.lax.broadcasted_iota(jnp.int32, (s, bn), 1)
        pt = jnp.logical_and(pt, col < n_total)

    m_old = m_ref[:]
    blk_max = jnp.max(jnp.where(pt, gt, _NEG), axis=1, keepdims=True)
    m_new = jnp.maximum(m_old, blk_max)
    scale = jnp.exp(m_old - m_new)
    e = jnp.where(pt, jnp.exp(gt - m_new), 0.0)
    d_ref[:] = d_ref[:] * scale + jnp.sum(e, axis=1, keepdims=True)
    m_ref[:] = m_new
    out_ref[:] = (out_ref[:] * scale
                  + jnp.dot(e.astype(jnp.bfloat16), xb,
                            preferred_element_type=jnp.float32))

    @pl.when(i == nsteps - 1)
    def _fin():
        out_ref[:] = out_ref[:] / (d_ref[:] + 1e-16)


def kernel(inputs, batch, W1, b1, W2, b2):
    n, d_dim = inputs.shape
    h_dim = W1.shape[1]
    s = _S
    bn = 4000 if n % 4000 == 0 else (2000 if n % 2000 == 0 else 1000)
    pad = (-n) % bn
    if pad:
        inputs = jnp.pad(inputs, ((0, pad), (0, 0)))
        batch = jnp.pad(batch, (0, pad))
    nsteps = (n + pad) // bn

    batch3 = batch.astype(jnp.int32).reshape(nsteps, 1, bn)
    b1r = b1.reshape(1, h_dim).astype(jnp.bfloat16)
    w2r = W2.reshape(1, h_dim).astype(jnp.bfloat16)
    b2r = b2.reshape(1, 1)
    W1 = W1.astype(jnp.bfloat16)

    out = pl.pallas_call(
        functools.partial(_body, nsteps=nsteps, n_total=n, bn=bn, s=s,
                          padded=bool(pad)),
        grid=(nsteps,),
        in_specs=[
            pl.BlockSpec(memory_space=pl.ANY),
            pl.BlockSpec((1, 1, bn), lambda i: (i, 0, 0)),
            pl.BlockSpec((d_dim, h_dim), lambda i: (0, 0)),
            pl.BlockSpec((1, h_dim), lambda i: (0, 0)),
            pl.BlockSpec((1, h_dim), lambda i: (0, 0)),
            pl.BlockSpec((1, 1), lambda i: (0, 0)),
        ],
        out_specs=pl.BlockSpec((s, d_dim), lambda i: (0, 0)),
        out_shape=jax.ShapeDtypeStruct((s, d_dim), jnp.float32),
        scratch_shapes=[pltpu.VMEM((s, 1), jnp.float32),
                        pltpu.VMEM((s, 1), jnp.float32),
                        pltpu.VMEM((2, bn, d_dim), jnp.float32),
                        pltpu.SemaphoreType.DMA((2,))],
        compiler_params=pltpu.CompilerParams(
            dimension_semantics=("arbitrary",)),
    )(inputs, batch3, W1, b1r, w2r, b2r)
    return out


# speedup vs baseline: 1.1125x; 1.0087x over previous
import functools

import jax
import jax.numpy as jnp
from jax.experimental import pallas as pl
from jax.experimental.pallas import tpu as pltpu

_S = 64
_NEG = -1e30


def _body(x_ref, b_ref, w1_ref, b1_ref, w2_ref, b2_ref, out_ref,
          m_ref, d_ref, *, nsteps, n_total, bn, s, padded):
    i = pl.program_id(0)

    @pl.when(i == 0)
    def _init():
        m_ref[:] = jnp.full((s, 1), _NEG, jnp.float32)
        d_ref[:] = jnp.zeros((s, 1), jnp.float32)
        out_ref[:] = jnp.zeros_like(out_ref)

    xb = x_ref[:].astype(jnp.bfloat16)
    h = jnp.maximum(
        jnp.dot(xb, w1_ref[:],
                preferred_element_type=jnp.float32).astype(jnp.bfloat16)
        + b1_ref[:], 0.0)
    gt = jax.lax.dot_general(w2_ref[:], h, (((1,), (1,)), ((), ())),
                             preferred_element_type=jnp.float32)
    gt = gt + b2_ref[0, 0]

    b_ids = b_ref[:].reshape(1, bn)
    seg = jax.lax.broadcasted_iota(jnp.int32, (s, bn), 0)
    pt = seg == b_ids
    if padded:
        col = i * bn + jax.lax.broadcasted_iota(jnp.int32, (s, bn), 1)
        pt = jnp.logical_and(pt, col < n_total)

    m_old = m_ref[:]
    blk_max = jnp.max(jnp.where(pt, gt, _NEG), axis=1, keepdims=True)
    m_new = jnp.maximum(m_old, blk_max)
    scale = jnp.exp(m_old - m_new)
    e = jnp.where(pt, jnp.exp(gt - m_new), 0.0)
    d_ref[:] = d_ref[:] * scale + jnp.sum(e, axis=1, keepdims=True)
    m_ref[:] = m_new
    out_ref[:] = (out_ref[:] * scale
                  + jnp.dot(e.astype(jnp.bfloat16), xb,
                            preferred_element_type=jnp.float32))

    @pl.when(i == nsteps - 1)
    def _fin():
        out_ref[:] = out_ref[:] / (d_ref[:] + 1e-16)


def kernel(inputs, batch, W1, b1, W2, b2):
    n, d_dim = inputs.shape
    h_dim = W1.shape[1]
    s = _S
    bn = 4000 if n % 4000 == 0 else (2000 if n % 2000 == 0 else 1000)
    pad = (-n) % bn
    if pad:
        inputs = jnp.pad(inputs, ((0, pad), (0, 0)))
        batch = jnp.pad(batch, (0, pad))
    nsteps = (n + pad) // bn

    batch3 = batch.astype(jnp.int32).reshape(nsteps, 1, bn)
    b1r = b1.reshape(1, h_dim).astype(jnp.bfloat16)
    w2r = W2.reshape(1, h_dim).astype(jnp.bfloat16)
    b2r = b2.reshape(1, 1)
    W1 = W1.astype(jnp.bfloat16)

    out = pl.pallas_call(
        functools.partial(_body, nsteps=nsteps, n_total=n, bn=bn, s=s,
                          padded=bool(pad)),
        grid=(nsteps,),
        in_specs=[
            pl.BlockSpec((bn, d_dim), lambda i: (i, 0)),
            pl.BlockSpec((1, 1, bn), lambda i: (i, 0, 0)),
            pl.BlockSpec((d_dim, h_dim), lambda i: (0, 0)),
            pl.BlockSpec((1, h_dim), lambda i: (0, 0)),
            pl.BlockSpec((1, h_dim), lambda i: (0, 0)),
            pl.BlockSpec((1, 1), lambda i: (0, 0)),
        ],
        out_specs=pl.BlockSpec((s, d_dim), lambda i: (0, 0)),
        out_shape=jax.ShapeDtypeStruct((s, d_dim), jnp.float32),
        scratch_shapes=[pltpu.VMEM((s, 1), jnp.float32),
                        pltpu.VMEM((s, 1), jnp.float32)],
        compiler_params=pltpu.CompilerParams(
            dimension_semantics=("arbitrary",)),
    )(inputs, batch3, W1, b1r, w2r, b2r)
    return out


# speedup vs baseline: 1.1169x; 1.0040x over previous
import functools

import jax
import jax.numpy as jnp
from jax.experimental import pallas as pl
from jax.experimental.pallas import tpu as pltpu

_S = 64
_NEG = -1e30


def _body(x_ref, b_ref, w1_ref, b1_ref, w2_ref, b2_ref, out_ref,
          m_ref, d_ref, *, nsteps, n_total, bn, s, padded):
    i = pl.program_id(0)

    @pl.when(i == 0)
    def _init():
        m_ref[:] = jnp.full((s, 1), _NEG, jnp.float32)
        d_ref[:] = jnp.zeros((s, 1), jnp.float32)
        out_ref[:] = jnp.zeros_like(out_ref)

    xb = x_ref[:].astype(jnp.bfloat16)
    h = jnp.maximum(
        jnp.dot(xb, w1_ref[:],
                preferred_element_type=jnp.float32).astype(jnp.bfloat16)
        + b1_ref[:], 0.0)
    gt = jax.lax.dot_general(w2_ref[:], h, (((1,), (1,)), ((), ())),
                             preferred_element_type=jnp.float32)
    gt = gt + b2_ref[0, 0]

    b_ids = b_ref[:].reshape(1, bn)
    seg = jax.lax.broadcasted_iota(jnp.int32, (s, bn), 0)
    pt = seg == b_ids
    if padded:
        col = i * bn + jax.lax.broadcasted_iota(jnp.int32, (s, bn), 1)
        pt = jnp.logical_and(pt, col < n_total)

    m_old = m_ref[:]
    blk_max = jnp.max(jnp.where(pt, gt, _NEG), axis=1, keepdims=True)
    m_new = jnp.maximum(m_old, blk_max)
    scale = jnp.exp(m_old - m_new)
    e = jnp.where(pt, jnp.exp(gt - m_new), 0.0)
    d_ref[:] = d_ref[:] * scale + jnp.sum(e, axis=1, keepdims=True)
    m_ref[:] = m_new
    out_ref[:] = (out_ref[:] * scale
                  + jnp.dot(e.astype(jnp.bfloat16), xb,
                            preferred_element_type=jnp.float32))

    @pl.when(i == nsteps - 1)
    def _fin():
        out_ref[:] = out_ref[:] / (d_ref[:] + 1e-16)


def kernel(inputs, batch, W1, b1, W2, b2):
    n, d_dim = inputs.shape
    h_dim = W1.shape[1]
    s = _S
    bn = 5000 if n % 5000 == 0 else (2000 if n % 2000 == 0 else 1000)
    pad = (-n) % bn
    if pad:
        inputs = jnp.pad(inputs, ((0, pad), (0, 0)))
        batch = jnp.pad(batch, (0, pad))
    nsteps = (n + pad) // bn

    batch3 = batch.astype(jnp.int32).reshape(nsteps, 1, bn)
    b1r = b1.reshape(1, h_dim).astype(jnp.bfloat16)
    w2r = W2.reshape(1, h_dim).astype(jnp.bfloat16)
    b2r = b2.reshape(1, 1)
    W1 = W1.astype(jnp.bfloat16)

    out = pl.pallas_call(
        functools.partial(_body, nsteps=nsteps, n_total=n, bn=bn, s=s,
                          padded=bool(pad)),
        grid=(nsteps,),
        in_specs=[
            pl.BlockSpec((bn, d_dim), lambda i: (i, 0)),
            pl.BlockSpec((1, 1, bn), lambda i: (i, 0, 0)),
            pl.BlockSpec((d_dim, h_dim), lambda i: (0, 0)),
            pl.BlockSpec((1, h_dim), lambda i: (0, 0)),
            pl.BlockSpec((1, h_dim), lambda i: (0, 0)),
            pl.BlockSpec((1, 1), lambda i: (0, 0)),
        ],
        out_specs=pl.BlockSpec((s, d_dim), lambda i: (0, 0)),
        out_shape=jax.ShapeDtypeStruct((s, d_dim), jnp.float32),
        scratch_shapes=[pltpu.VMEM((s, 1), jnp.float32),
                        pltpu.VMEM((s, 1), jnp.float32)],
        compiler_params=pltpu.CompilerParams(
            dimension_semantics=("arbitrary",)),
    )(inputs, batch3, W1, b1r, w2r, b2r)
    return out
